# Initial kernel scaffold; baseline (speedup 1.0000x reference)
import jax
import jax.numpy as jnp
from jax.experimental import pallas as pl


def kernel(z, W0, b0, A0, B0, gat_lin, gat_asrc, gat_adst, gat_bias, Wn, bn, An, Bn, Wa, ba, Aa, Ba, Wb, bb, Ab, Bb):
    raise NotImplementedError("write your pallas kernel here")


# speedup vs baseline: 260.2534x; 260.2534x over previous
import functools

import jax
import jax.numpy as jnp
from jax.experimental import pallas as pl

B = 256
LATENT = 128
HIDDEN = 256
MAXN = 32
NF = 16
LAYERS = 3
HEADS = 8
OUTH = HIDDEN // HEADS
R = 8
SCALING = 1.0 / R
GAT_IN = HIDDEN + LATENT
N = B * MAXN

G = 8
ROWS = G * MAXN
GRID = B // G

NT = 1024


def _init_kernel(z_ref, w0_ref, b0_ref, a0_ref, bb0_ref, out_ref):
    weff = w0_ref[...] + SCALING * jnp.dot(
        bb0_ref[...], a0_ref[...], preferred_element_type=jnp.float32)
    acc = jax.lax.dot_general(
        z_ref[...], weff, (((1,), (1,)), ((), ())),
        preferred_element_type=jnp.float32)
    out_ref[...] = jax.nn.relu(acc + b0_ref[...])


def _abt(a, b):
    return jax.lax.dot_general(a, b, (((1,), (1,)), ((), ())),
                               preferred_element_type=jnp.float32)


def _main_kernel(x0_ref, z_ref, lin_ref, asrc_ref, adst_ref, gbias_ref,
                 wn_ref, bn_ref, an_ref, bnn_ref,
                 wa_ref, ba_ref, aa_ref, baa_ref,
                 wb_ref, bb_ref, ab_ref, bab_ref,
                 recon_ref, adj_ref, bond_ref):
    f32 = jnp.float32
    r_iota = jax.lax.broadcasted_iota(jnp.int32, (ROWS, ROWS), 0)
    c_iota = jax.lax.broadcasted_iota(jnp.int32, (ROWS, ROWS), 1)
    bmat = (r_iota // MAXN == c_iota // MAXN).astype(f32)
    q32 = jax.lax.broadcasted_iota(jnp.int32, (ROWS, MAXN), 0)
    j32 = jax.lax.broadcasted_iota(jnp.int32, (ROWS, MAXN), 1)
    m32 = (q32 % MAXN == j32).astype(f32)
    imod = q32 % MAXN
    rep8 = (jax.lax.broadcasted_iota(jnp.int32, (ROWS, G), 0) // MAXN ==
            jax.lax.broadcasted_iota(jnp.int32, (ROWS, G), 1)).astype(f32)
    ind8 = (jax.lax.broadcasted_iota(jnp.int32, (HIDDEN, HEADS), 0) // OUTH ==
            jax.lax.broadcasted_iota(jnp.int32, (HIDDEN, HEADS), 1)).astype(f32)
    t32 = (jax.lax.broadcasted_iota(jnp.int32, (MAXN, ROWS), 1) % MAXN ==
           jax.lax.broadcasted_iota(jnp.int32, (MAXN, ROWS), 0)).astype(f32)

    z_blk = z_ref[...]
    z_rep = jnp.dot(rep8, z_blk, preferred_element_type=f32)

    x = x0_ref[...]
    for l in range(LAYERS):
        xc = jnp.concatenate([x, z_rep], axis=1)
        h = _abt(xc, lin_ref[l])
        a_s = jnp.dot(h * asrc_ref[l], ind8, preferred_element_type=f32)
        a_d = jnp.dot(h * adst_ref[l], ind8, preferred_element_type=f32)
        outs = []
        for hd in range(HEADS):
            as_col = a_s[:, hd:hd + 1]
            ad_col = a_d[:, hd:hd + 1]
            as_c = jnp.dot(bmat, as_col * m32, preferred_element_type=f32)
            s = ad_col + as_c
            s = jnp.where(s >= 0, s, 0.2 * s)
            m = jnp.max(s, axis=1, keepdims=True)
            e = jnp.exp(s - m)
            den = jnp.sum(e, axis=1, keepdims=True) + 1e-16
            attn = e / den
            af = jnp.dot(attn, t32, preferred_element_type=f32) * bmat
            outs.append(jnp.dot(af, h[:, hd * OUTH:(hd + 1) * OUTH],
                                preferred_element_type=f32))
        out = jnp.concatenate(outs, axis=1)
        x = jax.nn.relu(out + gbias_ref[l])

    wneff = wn_ref[...] + SCALING * jnp.dot(
        bnn_ref[...], an_ref[...], preferred_element_type=f32)
    recon_ref[...] = _abt(x, wneff) + bn_ref[...]

    waeff = wa_ref[...] + SCALING * jnp.dot(
        baa_ref[...], aa_ref[...], preferred_element_type=f32)
    u_col = _abt(x, waeff[:, :HIDDEN])
    v_col = _abt(x, waeff[:, HIDDEN:])
    vmat = jnp.dot(bmat, v_col * m32, preferred_element_type=f32)
    umat = jnp.dot(bmat, u_col * m32, preferred_element_type=f32)
    ba_s = ba_ref[0, 0]
    upper = (imod < j32)
    lower = (imod > j32)
    adj_ref[...] = (jnp.where(upper, u_col + vmat + ba_s, 0.0) +
                    jnp.where(lower, umat + v_col + ba_s, 0.0))

    wbeff = wb_ref[...] + SCALING * jnp.dot(
        bab_ref[...], ab_ref[...], preferred_element_type=f32)
    uc4 = _abt(x, wbeff[:, :HIDDEN])
    vc4 = _abt(x, wbeff[:, HIDDEN:])
    nb = 4 * MAXN
    tile4 = (jax.lax.broadcasted_iota(jnp.int32, (4, nb), 1) % 4 ==
             jax.lax.broadcasted_iota(jnp.int32, (4, nb), 0)).astype(f32)
    u_t = jnp.dot(uc4, tile4, preferred_element_type=f32)
    v_t = jnp.dot(vc4, tile4, preferred_element_type=f32)
    cb = jax.lax.broadcasted_iota(jnp.int32, (ROWS, nb), 1)
    qb = jax.lax.broadcasted_iota(jnp.int32, (ROWS, nb), 0)
    sel = (qb % MAXN == cb // 4).astype(f32)
    vexp = jnp.dot(bmat, v_t * sel, preferred_element_type=f32)
    uexp = jnp.dot(bmat, u_t * sel, preferred_element_type=f32)
    bb_t = jnp.dot(bb_ref[...], tile4, preferred_element_type=f32)
    iblk = qb % MAXN
    jblk = cb // 4
    bond_ref[...] = (
        jnp.where(iblk < jblk, u_t + vexp + bb_t, 0.0) +
        jnp.where(iblk > jblk, uexp + v_t + bb_t, 0.0))


@jax.jit
def kernel(z, W0, b0, A0, B0, gat_lin, gat_asrc, gat_adst, gat_bias,
           Wn, bn, An, Bn, Wa, ba, Aa, Ba, Wb, bb, Ab, Bb):
    f32 = jnp.float32
    nodes = pl.pallas_call(
        _init_kernel,
        grid=(HIDDEN * MAXN // NT,),
        in_specs=[
            pl.BlockSpec((B, LATENT), lambda j: (0, 0)),
            pl.BlockSpec((NT, LATENT), lambda j: (j, 0)),
            pl.BlockSpec((1, NT), lambda j: (0, j)),
            pl.BlockSpec((R, LATENT), lambda j: (0, 0)),
            pl.BlockSpec((NT, R), lambda j: (j, 0)),
        ],
        out_specs=pl.BlockSpec((B, NT), lambda j: (0, j)),
        out_shape=jax.ShapeDtypeStruct((B, HIDDEN * MAXN), f32),
    )(z, W0, b0.reshape(1, -1), A0, B0)
    x0 = nodes.reshape(N, HIDDEN)

    full = lambda *s: pl.BlockSpec(s, lambda p: tuple(0 for _ in s))
    recon, adjf, bondf = pl.pallas_call(
        _main_kernel,
        grid=(GRID,),
        in_specs=[
            pl.BlockSpec((ROWS, HIDDEN), lambda p: (p, 0)),
            pl.BlockSpec((G, LATENT), lambda p: (p, 0)),
            full(LAYERS, HIDDEN, GAT_IN),
            full(LAYERS, 1, HIDDEN),
            full(LAYERS, 1, HIDDEN),
            full(LAYERS, 1, HIDDEN),
            full(NF, HIDDEN), full(1, NF), full(R, HIDDEN), full(NF, R),
            full(1, 2 * HIDDEN), full(1, 1), full(R, 2 * HIDDEN), full(1, R),
            full(4, 2 * HIDDEN), full(1, 4), full(R, 2 * HIDDEN), full(4, R),
        ],
        out_specs=[
            pl.BlockSpec((ROWS, NF), lambda p: (p, 0)),
            pl.BlockSpec((ROWS, MAXN), lambda p: (p, 0)),
            pl.BlockSpec((ROWS, 4 * MAXN), lambda p: (p, 0)),
        ],
        out_shape=[
            jax.ShapeDtypeStruct((N, NF), f32),
            jax.ShapeDtypeStruct((N, MAXN), f32),
            jax.ShapeDtypeStruct((N, 4 * MAXN), f32),
        ],
    )(x0, z,
      gat_lin, gat_asrc.reshape(LAYERS, 1, HIDDEN),
      gat_adst.reshape(LAYERS, 1, HIDDEN), gat_bias.reshape(LAYERS, 1, HIDDEN),
      Wn, bn.reshape(1, NF), An, Bn,
      Wa, ba.reshape(1, 1), Aa, Ba,
      Wb, bb.reshape(1, 4), Ab, Bb)

    return (recon.reshape(B, MAXN, NF),
            adjf.reshape(B, MAXN, MAXN),
            bondf.reshape(B, MAXN, MAXN, 4))


# speedup vs baseline: 349.8365x; 1.3442x over previous
import jax
import jax.numpy as jnp
from jax.experimental import pallas as pl
from jax.experimental.pallas import tpu as pltpu

B = 256
LATENT = 128
HIDDEN = 256
MAXN = 32
NF = 16
LAYERS = 3
HEADS = 8
OUTH = HIDDEN // HEADS
R = 8
SCALING = 1.0 / R
GAT_IN = HIDDEN + LATENT
N = B * MAXN

G = 8
ROWS = G * MAXN
GRID = B // G

NT = 1024


def _prep_kernel(wn_ref, an_ref, bnn_ref, wa_ref, aa_ref, baa_ref,
                 wb_ref, ab_ref, bab_ref, wne_ref, wae_ref, wbe_ref):
    f32 = jnp.float32
    wne_ref[...] = wn_ref[...] + SCALING * jnp.dot(
        bnn_ref[...], an_ref[...], preferred_element_type=f32)
    wae_ref[...] = wa_ref[...] + SCALING * jnp.dot(
        baa_ref[...], aa_ref[...], preferred_element_type=f32)
    wbe_ref[...] = wb_ref[...] + SCALING * jnp.dot(
        bab_ref[...], ab_ref[...], preferred_element_type=f32)


def _init_kernel(z_ref, w0_ref, b0_ref, a0_ref, bb0_ref, out_ref):
    weff = w0_ref[...] + SCALING * jnp.dot(
        bb0_ref[...], a0_ref[...], preferred_element_type=jnp.float32)
    acc = jax.lax.dot_general(
        z_ref[...], weff, (((1,), (1,)), ((), ())),
        preferred_element_type=jnp.float32)
    out_ref[...] = jax.nn.relu(acc + b0_ref[...])


def _abt(a, b):
    return jax.lax.dot_general(a, b, (((1,), (1,)), ((), ())),
                               preferred_element_type=jnp.float32)


def _dot(a, b):
    return jnp.dot(a, b, preferred_element_type=jnp.float32)


def _main_kernel(x0_ref, z_ref, lin_ref, asrc_ref, adst_ref, gbias_ref,
                 wne_ref, bn_ref, wae_ref, ba_ref, wbe_ref, bb_ref,
                 hmask_ref, sel32_ref, m32_ref, rep8_ref, grep8_ref,
                 ind8_ref, exp8_ref, uadj_ref, ladj_ref,
                 tile4_ref, selb_ref, ubond_ref, lbond_ref,
                 recon_ref, adj_ref, bond_ref):
    hmask = hmask_ref[...]
    sel32 = sel32_ref[...]
    m32 = m32_ref[...]
    rep8 = rep8_ref[...]
    grep8 = grep8_ref[...]
    ind8 = ind8_ref[...]
    exp8 = exp8_ref[...]

    z_rep = _dot(rep8, z_ref[...])

    x = x0_ref[...]
    for l in range(LAYERS):
        xc = jnp.concatenate([x, z_rep], axis=1)
        h = _abt(xc, lin_ref[l])
        a_s = _dot(h * asrc_ref[l], ind8)
        a_d = _dot(h * adst_ref[l], ind8)
        ad_b = _dot(a_d, exp8)
        dall = _dot(a_s, exp8) * sel32
        as_all = _dot(rep8, _dot(grep8, dall))
        s = ad_b + as_all
        s = jnp.where(s >= 0, s, 0.2 * s)
        m = jnp.max(s, axis=1, keepdims=True)
        e = jnp.exp(s - m)
        rden8 = 1.0 / (_dot(e, ind8) + 1e-16)
        attn = e * _dot(rden8, exp8)
        outs = []
        for g in range(G):
            h_g = h[g * MAXN:(g + 1) * MAXN, :]
            hbd = _dot(m32, h_g) * hmask
            outs.append(_dot(attn[g * MAXN:(g + 1) * MAXN, :], hbd))
        out = jnp.concatenate(outs, axis=0)
        x = jax.nn.relu(out + gbias_ref[l])

    recon_ref[...] = _abt(x, wne_ref[...]) + bn_ref[...]

    waeff = wae_ref[...]
    u_col = _abt(x, waeff[:, :HIDDEN])
    v_col = _abt(x, waeff[:, HIDDEN:])
    vmat = _dot(rep8, _dot(grep8, v_col * m32))
    umat = _dot(rep8, _dot(grep8, u_col * m32))
    ba_s = ba_ref[0, 0]
    adj_ref[...] = (uadj_ref[...] * (u_col + vmat + ba_s) +
                    ladj_ref[...] * (umat + v_col + ba_s))

    wbeff = wbe_ref[...]
    uc4 = _abt(x, wbeff[:, :HIDDEN])
    vc4 = _abt(x, wbeff[:, HIDDEN:])
    tile4 = tile4_ref[...]
    selb = selb_ref[...]
    u_t = _dot(uc4, tile4)
    v_t = _dot(vc4, tile4)
    vexp = _dot(rep8, _dot(grep8, v_t * selb))
    uexp = _dot(rep8, _dot(grep8, u_t * selb))
    bb_t = _dot(bb_ref[...], tile4)
    bond_ref[...] = (ubond_ref[...] * (u_t + vexp + bb_t) +
                     lbond_ref[...] * (uexp + v_t + bb_t))


def _masks():
    f32 = jnp.float32
    ii = lambda shape, d: jax.lax.broadcasted_iota(jnp.int32, shape, d)
    hmask = (ii((HIDDEN, HIDDEN), 0) // OUTH ==
             ii((HIDDEN, HIDDEN), 1) // OUTH).astype(f32)
    sel32 = (ii((ROWS, HIDDEN), 0) % MAXN ==
             ii((ROWS, HIDDEN), 1) % MAXN).astype(f32)
    m32 = (ii((ROWS, MAXN), 0) % MAXN == ii((ROWS, MAXN), 1)).astype(f32)
    rep8 = (ii((ROWS, G), 0) // MAXN == ii((ROWS, G), 1)).astype(f32)
    grep8 = (ii((G, ROWS), 0) == ii((G, ROWS), 1) // MAXN).astype(f32)
    ind8 = (ii((HIDDEN, HEADS), 0) // OUTH == ii((HIDDEN, HEADS), 1)).astype(f32)
    exp8 = (ii((HEADS, HIDDEN), 0) == ii((HEADS, HIDDEN), 1) // OUTH).astype(f32)
    imod = ii((ROWS, MAXN), 0) % MAXN
    j32 = ii((ROWS, MAXN), 1)
    uadj = (imod < j32).astype(f32)
    ladj = (imod > j32).astype(f32)
    nb = 4 * MAXN
    tile4 = (ii((4, nb), 1) % 4 == ii((4, nb), 0)).astype(f32)
    qb = ii((ROWS, nb), 0) % MAXN
    cb = ii((ROWS, nb), 1) // 4
    selb = (qb == cb).astype(f32)
    ubond = (qb < cb).astype(f32)
    lbond = (qb > cb).astype(f32)
    return hmask, sel32, m32, rep8, grep8, ind8, exp8, uadj, ladj, tile4, selb, ubond, lbond


@jax.jit
def kernel(z, W0, b0, A0, B0, gat_lin, gat_asrc, gat_adst, gat_bias,
           Wn, bn, An, Bn, Wa, ba, Aa, Ba, Wb, bb, Ab, Bb):
    f32 = jnp.float32
    wne, wae, wbe = pl.pallas_call(
        _prep_kernel,
        out_shape=[
            jax.ShapeDtypeStruct((NF, HIDDEN), f32),
            jax.ShapeDtypeStruct((1, 2 * HIDDEN), f32),
            jax.ShapeDtypeStruct((4, 2 * HIDDEN), f32),
        ],
    )(Wn, An, Bn, Wa, Aa, Ba, Wb, Ab, Bb)

    nodes = pl.pallas_call(
        _init_kernel,
        grid=(HIDDEN * MAXN // NT,),
        in_specs=[
            pl.BlockSpec((B, LATENT), lambda j: (0, 0)),
            pl.BlockSpec((NT, LATENT), lambda j: (j, 0)),
            pl.BlockSpec((1, NT), lambda j: (0, j)),
            pl.BlockSpec((R, LATENT), lambda j: (0, 0)),
            pl.BlockSpec((NT, R), lambda j: (j, 0)),
        ],
        out_specs=pl.BlockSpec((B, NT), lambda j: (0, j)),
        out_shape=jax.ShapeDtypeStruct((B, HIDDEN * MAXN), f32),
        compiler_params=pltpu.CompilerParams(
            dimension_semantics=("parallel",)),
    )(z, W0, b0.reshape(1, -1), A0, B0)
    x0 = nodes.reshape(N, HIDDEN)

    masks = _masks()
    full = lambda *s: pl.BlockSpec(s, lambda p: tuple(0 for _ in s))
    recon, adjf, bondf = pl.pallas_call(
        _main_kernel,
        grid=(GRID,),
        in_specs=[
            pl.BlockSpec((ROWS, HIDDEN), lambda p: (p, 0)),
            pl.BlockSpec((G, LATENT), lambda p: (p, 0)),
            full(LAYERS, HIDDEN, GAT_IN),
            full(LAYERS, 1, HIDDEN),
            full(LAYERS, 1, HIDDEN),
            full(LAYERS, 1, HIDDEN),
            full(NF, HIDDEN), full(1, NF),
            full(1, 2 * HIDDEN), full(1, 1),
            full(4, 2 * HIDDEN), full(1, 4),
        ] + [full(*m.shape) for m in masks],
        out_specs=[
            pl.BlockSpec((ROWS, NF), lambda p: (p, 0)),
            pl.BlockSpec((ROWS, MAXN), lambda p: (p, 0)),
            pl.BlockSpec((ROWS, 4 * MAXN), lambda p: (p, 0)),
        ],
        out_shape=[
            jax.ShapeDtypeStruct((N, NF), f32),
            jax.ShapeDtypeStruct((N, MAXN), f32),
            jax.ShapeDtypeStruct((N, 4 * MAXN), f32),
        ],
        compiler_params=pltpu.CompilerParams(
            dimension_semantics=("parallel",)),
    )(x0, z,
      gat_lin, gat_asrc.reshape(LAYERS, 1, HIDDEN),
      gat_adst.reshape(LAYERS, 1, HIDDEN), gat_bias.reshape(LAYERS, 1, HIDDEN),
      wne, bn.reshape(1, NF),
      wae, ba.reshape(1, 1),
      wbe, bb.reshape(1, 4),
      *masks)

    return (recon.reshape(B, MAXN, NF),
            adjf.reshape(B, MAXN, MAXN),
            bondf.reshape(B, MAXN, MAXN, 4))
